# baseline (speedup 1.0000x reference)
import functools

import jax
import jax.numpy as jnp
from jax import lax
from jax.experimental import pallas as pl
from jax.experimental.pallas import tpu as pltpu
from jax.experimental.pallas import tpu_sc as plsc

_EPS = 1e-15
_N_USERS = 2000
_N_ITEMS = 8000
_N = _N_USERS + _N_ITEMS
_D = 128
_DP = 128
_E = 320000
_G = 80
_NT = 16
_NGRP = 256
_EPAD = _NT * _NGRP * _G
_CHG = 16
_NCH = _NGRP // _CHG
_TRASH = 10200
_NPAD = 10240
_ROWS_PT = _NPAD // _NT
_ROWS_LAST = _N - (_NT - 1) * _ROWS_PT


def _masks(width=_D):
    l = lax.broadcasted_iota(jnp.int32, (1, width), 1)
    m = (l >= 1).astype(jnp.float32)
    return m, 1.0 - m


def _rn2(x):
    return jnp.sum(x * x, axis=-1, keepdims=True)


def _sinh(n):
    return 0.5 * (jnp.exp(n) - jnp.exp(-n))


def _arccosh(x):
    return jnp.log(x + jnp.sqrt(x * x - 1.0))


def _expmap0_projx(u, m, e0):
    us = u * m
    n = jnp.sqrt(jnp.clip(_rn2(us), 1e-12))
    xs = _sinh(n) * us / n
    x0 = jnp.sqrt(1.0 + _rn2(xs))
    return x0 * e0 + xs


def _logmap0(x, m):
    x0 = x[:, 0:1]
    xs = x * m
    d = _arccosh(jnp.clip(x0, 1.0 + 1e-7))
    n = jnp.sqrt(jnp.clip(_rn2(xs), 1e-12))
    return (d / n) * xs


def _prologue_body(emb_ref, T_ref, ugr_ref, sps_ref, W1_ref, W2_ref, tab_ref):
    m, e0 = _masks()
    x1 = _expmap0_projx(emb_ref[...], m, e0)
    t1 = jnp.dot(_logmap0(x1, m), W1_ref[...],
                 preferred_element_type=jnp.float32) * m
    emb_tag = _expmap0_projx(T_ref[...], m, e0)
    p = (emb_tag * m) / (emb_tag[:, 0:1] + 1.0)
    xk = 2.0 * p / (1.0 + _rn2(p))
    gamma = 1.0 / jnp.sqrt(jnp.clip(1.0 - _rn2(xk), _EPS))
    B = gamma * (xk + e0)
    mm = jnp.dot(sps_ref[...], B, preferred_element_type=jnp.float32)
    mean = (mm * m) / jnp.clip(mm[:, 0:1], _EPS)
    kp = mean / (1.0 + jnp.sqrt(jnp.clip(1.0 - _rn2(mean), _EPS)))
    pn = _rn2(kp)
    dnm = jnp.clip(1.0 - pn, _EPS)
    x2_out = ((1.0 + pn) / dnm) * e0 + (2.0 * kp) / dnm
    x2_in = _expmap0_projx(ugr_ref[...], m, e0)
    x2 = jnp.concatenate([x2_in, x2_out], axis=0)
    t2 = jnp.dot(_logmap0(x2, m), W2_ref[...],
                 preferred_element_type=jnp.float32) * m
    tab_ref[...] = jnp.concatenate([t1 + e0, t2 + e0], axis=0)


def _epilogue_body(aggs_ref, out_ref):
    m, e0 = _masks()

    def finish(a_pad):
        deg = jnp.maximum(a_pad[:, 0:1], 1.0)
        us = (a_pad * m) / deg
        n = jnp.sqrt(jnp.clip(_rn2(us), 1e-12))
        xs = _sinh(n) * us / n
        x0 = jnp.sqrt(1.0 + _rn2(xs))
        return x0 * e0 + xs

    h1 = finish(aggs_ref[0:_N, :])
    h2 = finish(aggs_ref[_N:2 * _N, :])
    out_ref[...] = jnp.concatenate([h1, h2], axis=-1)


def _sc_body(tab_ref, srcg_ref, dstg_ref, zrows_ref, out_ref,
             sidx, didx, rows, agg,
             gsem0, gsem1, gsem2, gsem3, ssem0, ssem1, ssem2, ssem3,
             isg, isd):
    gsem = (gsem0, gsem1, gsem2, gsem3)
    ssem = (ssem0, ssem1, ssem2, ssem3)
    c = lax.axis_index("c")
    s = lax.axis_index("s")
    sbase = c * (_EPAD // _G) + s * _NGRP
    dbase = s * _NGRP
    pltpu.sync_copy(zrows_ref, agg.at[pl.ds(s * _ROWS_PT, _ROWS_PT)])
    plsc.subcore_barrier()

    def gather(g, b):
        pltpu.async_copy(tab_ref.at[sidx.at[(g // _CHG) % 2, g % _CHG]],
                         rows.at[b], gsem[b])

    def scatter(g, b):
        pltpu.async_copy(rows.at[b], agg.at[didx.at[(g // _CHG) % 2,
                                                    g % _CHG]],
                         ssem[b], add=True)

    def wait_gather(b):
        pltpu.make_async_copy(tab_ref.at[sidx.at[0, 0]], rows.at[b],
                              gsem[b]).wait()

    def wait_scatter(b):
        pltpu.make_async_copy(rows.at[b], agg.at[didx.at[0, 0]],
                              ssem[b]).wait()

    pltpu.sync_copy(srcg_ref.at[pl.ds(sbase, _CHG)], sidx.at[0])
    pltpu.sync_copy(dstg_ref.at[pl.ds(dbase, _CHG)], didx.at[0])
    for g in range(3):
        gather(g, g)

    def body(i, carry):
        for b in range(4):
            g = i * 4 + b
            nb = (b + 3) % 4

            @pl.when(jnp.logical_and(g % _CHG == _CHG - 3,
                                     g < _NGRP - _CHG))
            def _():
                pltpu.make_async_copy(srcg_ref.at[pl.ds(sbase, _CHG)],
                                      sidx.at[0], isg).wait()
                pltpu.make_async_copy(dstg_ref.at[pl.ds(dbase, _CHG)],
                                      didx.at[0], isd).wait()

            @pl.when(g + 3 < _NGRP)
            def _():
                @pl.when(g >= 1)
                def _():
                    wait_scatter(nb)
                gather(g + 3, nb)

            @pl.when(jnp.logical_and(g % _CHG == 0, g < _NGRP - _CHG))
            def _():
                k1 = g // _CHG + 1
                pltpu.async_copy(
                    srcg_ref.at[pl.ds(sbase + k1 * _CHG, _CHG)],
                    sidx.at[k1 % 2], isg)
                pltpu.async_copy(
                    dstg_ref.at[pl.ds(dbase + k1 * _CHG, _CHG)],
                    didx.at[k1 % 2], isd)

            wait_gather(b)
            scatter(g, b)
        return carry

    lax.fori_loop(0, _NGRP // 4, body, 0)
    for b in range(4):
        wait_scatter(b)
    plsc.subcore_barrier()

    @pl.when(s < _NT - 1)
    def _():
        pltpu.sync_copy(agg.at[pl.ds(s * _ROWS_PT, _ROWS_PT)],
                        out_ref.at[pl.ds(c * _N + s * _ROWS_PT, _ROWS_PT)])

    @pl.when(s == _NT - 1)
    def _():
        pltpu.sync_copy(
            agg.at[pl.ds((_NT - 1) * _ROWS_PT, _ROWS_LAST)],
            out_ref.at[pl.ds(c * _N + (_NT - 1) * _ROWS_PT, _ROWS_LAST)])


def _make_sc_agg():
    return functools.partial(
        pl.kernel,
        out_type=jax.ShapeDtypeStruct((2 * _N, _DP), jnp.float32),
        mesh=plsc.VectorSubcoreMesh(core_axis_name="c", subcore_axis_name="s",
                                    num_cores=2, num_subcores=_NT),
        scratch_types=(
            [pltpu.VMEM((2, _CHG, _G), jnp.int32),
             pltpu.VMEM((2, _CHG, _G), jnp.int32),
             pltpu.VMEM((4, _G, _DP), jnp.float32),
             pltpu.VMEM_SHARED((_NPAD, _DP), jnp.float32)]
            + [pltpu.SemaphoreType.DMA] * 10),
    )(_sc_body)


def kernel(emb_weight, T_weight, ugr_weight, sps, W1, W2, edge_index):
    npad = _EPAD - _E
    src = jnp.concatenate([edge_index[0],
                           jnp.zeros((npad,), jnp.int32)])
    dst = jnp.concatenate([edge_index[1],
                           jnp.full((npad,), _TRASH, jnp.int32)])
    srcg = jnp.concatenate([src, src + _N]).reshape(2 * _EPAD // _G, _G)
    dstg = dst.reshape(_EPAD // _G, _G)
    zrows = jnp.zeros((_ROWS_PT, _DP), jnp.float32)

    tables = pl.pallas_call(
        _prologue_body,
        out_shape=jax.ShapeDtypeStruct((2 * _N, _DP), jnp.float32),
    )(emb_weight, T_weight, ugr_weight, sps, W1, W2)

    aggs = _make_sc_agg()(tables, srcg, dstg, zrows)

    return pl.pallas_call(
        _epilogue_body,
        out_shape=jax.ShapeDtypeStruct((_N, 2 * _D), jnp.float32),
    )(aggs)


# speedup vs baseline: 2.2073x; 2.2073x over previous
import functools

import jax
import jax.numpy as jnp
from jax import lax
from jax.experimental import pallas as pl
from jax.experimental.pallas import tpu as pltpu
from jax.experimental.pallas import tpu_sc as plsc

_EPS = 1e-15
_N_USERS = 2000
_N_ITEMS = 8000
_N = _N_USERS + _N_ITEMS
_D = 128
_DP = 128
_E = 320000
_G = 125
_NT = 16
_EPT = _E // _NT
_NGRP = _EPT // _G
_CHG = 40
_NPAD = 10240
_ROWS_PT = _NPAD // _NT
_ROWS_LAST = _N - (_NT - 1) * _ROWS_PT


def _masks(width=_D):
    l = lax.broadcasted_iota(jnp.int32, (1, width), 1)
    m = (l >= 1).astype(jnp.float32)
    return m, 1.0 - m


def _rn2(x):
    return jnp.sum(x * x, axis=-1, keepdims=True)


def _sinh(n):
    return 0.5 * (jnp.exp(n) - jnp.exp(-n))


def _arccosh(x):
    return jnp.log(x + jnp.sqrt(x * x - 1.0))


def _expmap0_projx(u, m, e0):
    us = u * m
    n = jnp.sqrt(jnp.clip(_rn2(us), 1e-12))
    xs = _sinh(n) * us / n
    x0 = jnp.sqrt(1.0 + _rn2(xs))
    return x0 * e0 + xs


def _logmap0(x, m):
    x0 = x[:, 0:1]
    xs = x * m
    d = _arccosh(jnp.clip(x0, 1.0 + 1e-7))
    n = jnp.sqrt(jnp.clip(_rn2(xs), 1e-12))
    return (d / n) * xs


def _prologue_body(emb_ref, T_ref, ugr_ref, sps_ref, W1_ref, W2_ref, tab_ref):
    m, e0 = _masks()
    x1 = _expmap0_projx(emb_ref[...], m, e0)
    t1 = jnp.dot(_logmap0(x1, m), W1_ref[...],
                 preferred_element_type=jnp.float32) * m
    emb_tag = _expmap0_projx(T_ref[...], m, e0)
    p = (emb_tag * m) / (emb_tag[:, 0:1] + 1.0)
    xk = 2.0 * p / (1.0 + _rn2(p))
    gamma = 1.0 / jnp.sqrt(jnp.clip(1.0 - _rn2(xk), _EPS))
    B = gamma * (xk + e0)
    mm = jnp.dot(sps_ref[...], B, preferred_element_type=jnp.float32)
    mean = (mm * m) / jnp.clip(mm[:, 0:1], _EPS)
    kp = mean / (1.0 + jnp.sqrt(jnp.clip(1.0 - _rn2(mean), _EPS)))
    pn = _rn2(kp)
    dnm = jnp.clip(1.0 - pn, _EPS)
    x2_out = ((1.0 + pn) / dnm) * e0 + (2.0 * kp) / dnm
    x2_in = _expmap0_projx(ugr_ref[...], m, e0)
    x2 = jnp.concatenate([x2_in, x2_out], axis=0)
    t2 = jnp.dot(_logmap0(x2, m), W2_ref[...],
                 preferred_element_type=jnp.float32) * m
    tab_ref[...] = jnp.concatenate([t1 + e0, t2 + e0], axis=0)


def _epilogue_body(aggs_ref, out_ref):
    m, e0 = _masks()

    def finish(a_pad):
        deg = jnp.maximum(a_pad[:, 0:1], 1.0)
        us = (a_pad * m) / deg
        n = jnp.sqrt(jnp.clip(_rn2(us), 1e-12))
        xs = _sinh(n) * us / n
        x0 = jnp.sqrt(1.0 + _rn2(xs))
        return x0 * e0 + xs

    h1 = finish(aggs_ref[0:_N, :])
    h2 = finish(aggs_ref[_N:2 * _N, :])
    out_ref[...] = jnp.concatenate([h1, h2], axis=-1)


def _sc_body(tab_ref, srcg_ref, dstg_ref, zrows_ref, out_ref,
             sidx, didx, rows, agg, gsem0, gsem1, ssem0, ssem1):
    gsem = (gsem0, gsem1)
    ssem = (ssem0, ssem1)
    c = lax.axis_index("c")
    s = lax.axis_index("s")
    pltpu.sync_copy(zrows_ref, agg.at[pl.ds(s * _ROWS_PT, _ROWS_PT)])
    plsc.subcore_barrier()

    def gather(r, b):
        pltpu.async_copy(tab_ref.at[sidx.at[r]], rows.at[b], gsem[b])

    def scatter(r, b):
        pltpu.async_copy(rows.at[b], agg.at[didx.at[r]], ssem[b], add=True)

    def wait_gather(b):
        pltpu.make_async_copy(tab_ref.at[sidx.at[0]], rows.at[b],
                              gsem[b]).wait()

    def wait_scatter(b):
        pltpu.make_async_copy(rows.at[b], agg.at[didx.at[0]],
                              ssem[b]).wait()

    def chunk(k, carry):
        pltpu.sync_copy(
            srcg_ref.at[pl.ds(c * (_E // _G) + s * _NGRP + k * _CHG, _CHG)],
            sidx)
        pltpu.sync_copy(dstg_ref.at[pl.ds(s * _NGRP + k * _CHG, _CHG)], didx)
        gather(0, 0)
        gather(1, 1)
        wait_gather(0)
        scatter(0, 0)

        def body(j, carry2):
            for b in range(2):
                r = j * 2 + 1 + b
                sb = (1 + b) % 2
                nb = b
                wait_scatter(nb)
                gather(r + 1, nb)
                wait_gather(sb)
                scatter(r, sb)
            return carry2

        lax.fori_loop(0, (_CHG - 2) // 2, body, 0)
        wait_gather(1)
        scatter(_CHG - 1, 1)
        for b in range(2):
            wait_scatter(b)
        return carry

    lax.fori_loop(0, _NGRP // _CHG, chunk, 0)
    plsc.subcore_barrier()

    @pl.when(s < _NT - 1)
    def _():
        pltpu.sync_copy(agg.at[pl.ds(s * _ROWS_PT, _ROWS_PT)],
                        out_ref.at[pl.ds(c * _N + s * _ROWS_PT, _ROWS_PT)])

    @pl.when(s == _NT - 1)
    def _():
        pltpu.sync_copy(
            agg.at[pl.ds((_NT - 1) * _ROWS_PT, _ROWS_LAST)],
            out_ref.at[pl.ds(c * _N + (_NT - 1) * _ROWS_PT, _ROWS_LAST)])


def _make_sc_agg():
    return functools.partial(
        pl.kernel,
        out_type=jax.ShapeDtypeStruct((2 * _N, _DP), jnp.float32),
        mesh=plsc.VectorSubcoreMesh(core_axis_name="c", subcore_axis_name="s",
                                    num_cores=2, num_subcores=_NT),
        scratch_types=(
            [pltpu.VMEM((_CHG, _G), jnp.int32),
             pltpu.VMEM((_CHG, _G), jnp.int32),
             pltpu.VMEM((2, _G, _DP), jnp.float32),
             pltpu.VMEM_SHARED((_NPAD, _DP), jnp.float32)]
            + [pltpu.SemaphoreType.DMA] * 4),
    )(_sc_body)


def kernel(emb_weight, T_weight, ugr_weight, sps, W1, W2, edge_index):
    src = edge_index[0]
    dst = edge_index[1]
    srcg = jnp.concatenate([src, src + _N]).reshape(2 * _E // _G, _G)
    dstg = dst.reshape(_E // _G, _G)
    zrows = jnp.zeros((_ROWS_PT, _DP), jnp.float32)

    tables = pl.pallas_call(
        _prologue_body,
        out_shape=jax.ShapeDtypeStruct((2 * _N, _DP), jnp.float32),
    )(emb_weight, T_weight, ugr_weight, sps, W1, W2)

    aggs = _make_sc_agg()(tables, srcg, dstg, zrows)

    return pl.pallas_call(
        _epilogue_body,
        out_shape=jax.ShapeDtypeStruct((_N, 2 * _D), jnp.float32),
    )(aggs)
